# Initial kernel scaffold; baseline (speedup 1.0000x reference)
import jax
import jax.numpy as jnp
from jax.experimental import pallas as pl


def kernel(users_rep, items_rep, edge_index):
    raise NotImplementedError("write your pallas kernel here")


# speedup vs baseline: 53.8906x; 53.8906x over previous
import functools

import jax
import jax.numpy as jnp
from jax import lax
from jax.experimental import pallas as pl
from jax.experimental.pallas import tpu as pltpu
from jax.experimental.pallas import tpu_sc as plsc

U = 25000
I = 25000
EMB = 32
E = 800000

P = 25088
NTAB = 2 * P
NSUB = 16
CHUNK = 128
NCHUNK = 392
EPT = CHUNK * NCHUNK
EPC = NSUB * EPT
ROWS_PT = P // NSUB
PAD_IDX = 25080
DEGW = 16
BLK = 56
NBLK = NCHUNK // BLK
NBUF = 8
DCH = 224
NDCH = ROWS_PT // DCH


def _sc_mesh():
    return plsc.VectorSubcoreMesh(core_axis_name="c", subcore_axis_name="s")


_SC_PARAMS = pltpu.CompilerParams(
    use_tc_tiling_on_sc=False, needs_layout_passes=False)


def _rsqrt16(d):
    i = plsc.bitcast(d, jnp.int32)
    i = jnp.int32(0x5F3759DF) - lax.shift_right_logical(i, 1)
    y = plsc.bitcast(i, jnp.float32)
    for _ in range(3):
        y = y * (1.5 - 0.5 * d * y * y)
    return y


@functools.partial(
    pl.kernel,
    out_type=pltpu.HBM((NTAB, DEGW), jnp.float32),
    mesh=_sc_mesh(),
    scratch_types=[
        pltpu.VMEM((NCHUNK, CHUNK), jnp.int32),
        pltpu.VMEM((CHUNK, DEGW), jnp.float32),
        pltpu.VMEM_SHARED((P, DEGW), jnp.float32),
        pltpu.SemaphoreType.DMA,
    ],
    compiler_params=_SC_PARAMS,
)
def _sc_degrees(rows_hbm, ones_hbm, zeros_hbm, out_hbm, ridx, ones_v, deg_sh,
                sem):
    c = lax.axis_index("c")
    s = lax.axis_index("s")
    w = c * NSUB + s
    base = s * ROWS_PT
    pltpu.sync_copy(zeros_hbm.at[pl.ds(base, ROWS_PT)],
                    deg_sh.at[pl.ds(base, ROWS_PT)])
    pltpu.sync_copy(ones_hbm, ones_v)
    pltpu.sync_copy(rows_hbm.at[pl.ds(w * NCHUNK, NCHUNK)], ridx)
    plsc.subcore_barrier()

    def body(j, carry):
        pltpu.async_copy(ones_v, deg_sh.at[ridx.at[j]], sem, add=True)

        @pl.when(j >= 3)
        def _():
            pltpu.make_async_copy(ones_v, deg_sh.at[ridx.at[j]], sem).wait()

        return carry

    lax.fori_loop(0, NCHUNK, body, 0)
    for _ in range(3):
        pltpu.make_async_copy(ones_v, deg_sh.at[ridx.at[0]], sem).wait()
    plsc.subcore_barrier()
    pltpu.sync_copy(deg_sh.at[pl.ds(base, ROWS_PT)],
                    out_hbm.at[pl.ds(c * P + base, ROWS_PT)])


def _make_sc_layer(inv):
    @functools.partial(
        pl.kernel,
        out_type=[
            pltpu.HBM((NTAB, EMB), jnp.float32),
            pltpu.HBM((NTAB, EMB), jnp.float32),
        ],
        mesh=_sc_mesh(),
        scratch_types=[
            pltpu.VMEM((BLK, CHUNK), jnp.int32),
            pltpu.VMEM((BLK, CHUNK), jnp.int32),
            [pltpu.VMEM((CHUNK, EMB), jnp.float32) for _ in range(NBUF)],
            pltpu.VMEM((DCH, EMB), jnp.float32),
            pltpu.VMEM((DCH, DEGW), jnp.float32),
            pltpu.VMEM((DCH, EMB), jnp.float32),
            pltpu.VMEM_SHARED((P, EMB), jnp.float32),
            pltpu.SemaphoreType.DMA,
            pltpu.SemaphoreType.DMA,
        ],
        compiler_params=_SC_PARAMS,
    )
    def _sc_layer(deg_hbm, prev_hbm, rows_hbm, cols_hbm, zeros_hbm,
                  reps_out, tmp_hbm,
                  ridx, cidx, gbufs, pbuf, dbuf, tbuf, acc_sh, sem, sem2):
        c = lax.axis_index("c")
        s = lax.axis_index("s")
        w = c * NSUB + s
        base = s * ROWS_PT
        own0 = c * P + base
        oth0 = (1 - c) * P + base
        pltpu.sync_copy(zeros_hbm.at[pl.ds(base, ROWS_PT)],
                        acc_sh.at[pl.ds(base, ROWS_PT)])

        def p1(i, carry):
            rg = oth0 + i * DCH
            pltpu.sync_copy(prev_hbm.at[pl.ds(rg, DCH)], pbuf)
            pltpu.sync_copy(deg_hbm.at[pl.ds(rg, DCH)], dbuf)

            def rowf(r, carry2):
                sv = _rsqrt16(dbuf[r, pl.ds(0, 16)])
                tbuf[r, pl.ds(0, 16)] = pbuf[r, pl.ds(0, 16)] * sv
                tbuf[r, pl.ds(16, 16)] = pbuf[r, pl.ds(16, 16)] * sv
                return carry2

            lax.fori_loop(0, DCH, rowf, 0)
            pltpu.sync_copy(tbuf, tmp_hbm.at[pl.ds(c * P + base + i * DCH, DCH)])
            return carry

        lax.fori_loop(0, NDCH, p1, 0)
        plsc.subcore_barrier()

        GA = NBUF - 2

        def blk_body(b, carry):
            off = w * NCHUNK + b * BLK
            pltpu.sync_copy(rows_hbm.at[pl.ds(off, BLK)], ridx)
            pltpu.sync_copy(cols_hbm.at[pl.ds(off, BLK)], cidx)
            for k in range(GA):
                pltpu.async_copy(tmp_hbm.at[cidx.at[k]], gbufs[k], sem)

            def grp_body(q, carry2):
                for k in range(NBUF):
                    j = q * NBUF + k
                    pltpu.make_async_copy(
                        tmp_hbm.at[cidx.at[j]], gbufs[k], sem).wait()
                    pltpu.async_copy(
                        gbufs[k], acc_sh.at[ridx.at[j]], sem2, add=True)

                    @pl.when(j >= 2)
                    def _():
                        pltpu.make_async_copy(
                            gbufs[k], acc_sh.at[ridx.at[j]], sem2).wait()

                    @pl.when(j + GA < BLK)
                    def _():
                        pltpu.async_copy(
                            tmp_hbm.at[cidx.at[j + GA]],
                            gbufs[(k + GA) % NBUF], sem)
                return carry2

            lax.fori_loop(0, BLK // NBUF, grp_body, 0)
            for _ in range(2):
                pltpu.make_async_copy(
                    gbufs[0], acc_sh.at[ridx.at[0]], sem2).wait()
            return carry

        lax.fori_loop(0, NBLK, blk_body, 0)
        plsc.subcore_barrier()

        def p3(i, carry):
            rl = base + i * DCH
            rg = own0 + i * DCH
            pltpu.sync_copy(acc_sh.at[pl.ds(rl, DCH)], pbuf)
            pltpu.sync_copy(deg_hbm.at[pl.ds(rg, DCH)], dbuf)

            def rowf(r, carry2):
                sv = _rsqrt16(dbuf[r, pl.ds(0, 16)]) * inv
                tbuf[r, pl.ds(0, 16)] = pbuf[r, pl.ds(0, 16)] * sv
                tbuf[r, pl.ds(16, 16)] = pbuf[r, pl.ds(16, 16)] * sv
                return carry2

            lax.fori_loop(0, DCH, rowf, 0)
            pltpu.sync_copy(tbuf, reps_out.at[pl.ds(rg, DCH)])
            return carry

        lax.fori_loop(0, NDCH, p3, 0)

    return _sc_layer


_sc_layer1 = _make_sc_layer(0.5)
_sc_layer2 = _make_sc_layer(1.0 / 3.0)


TC_BS = 3136
_TC_GRID = NTAB // TC_BS


def _bspec(width):
    return pl.BlockSpec((TC_BS, width), lambda i: (i, 0))


def _tc_norm_body(acc_ref, reps_ref, out_ref):
    v = reps_ref[:]
    nrm = jnp.sqrt(jnp.sum(v * v, axis=1, keepdims=True))
    out_ref[:] = acc_ref[:] + v / jnp.maximum(nrm, 1e-12)


def _tc_norm(acc, reps):
    return pl.pallas_call(
        _tc_norm_body,
        grid=(_TC_GRID,),
        in_specs=[_bspec(EMB), _bspec(EMB)],
        out_specs=_bspec(EMB),
        out_shape=jax.ShapeDtypeStruct((NTAB, EMB), jnp.float32),
    )(acc, reps)


def _pad_edges(x):
    return jnp.concatenate(
        [x, jnp.full((EPC - E,), PAD_IDX, dtype=jnp.int32)])


def kernel(users_rep, items_rep, edge_index):
    src = edge_index[0].astype(jnp.int32)
    dst = edge_index[1].astype(jnp.int32)

    rows_w = jnp.concatenate([
        _pad_edges(src).reshape(NSUB * NCHUNK, CHUNK),
        _pad_edges(dst).reshape(NSUB * NCHUNK, CHUNK),
    ], axis=0)
    cols_w = jnp.concatenate([
        _pad_edges(dst).reshape(NSUB * NCHUNK, CHUNK),
        _pad_edges(src + P).reshape(NSUB * NCHUNK, CHUNK),
    ], axis=0)

    zpad = jnp.zeros((P - U, EMB), jnp.float32)
    reps0 = jnp.concatenate([users_rep, zpad, items_rep, zpad], axis=0)

    ones_hbm = jnp.ones((CHUNK, DEGW), jnp.float32)
    zeros_deg = jnp.zeros((P, DEGW), jnp.float32)
    zeros_tab = jnp.zeros((P, EMB), jnp.float32)

    deg_tab = _sc_degrees(rows_w, ones_hbm, zeros_deg)

    reps1, _ = _sc_layer1(deg_tab, reps0, rows_w, cols_w, zeros_tab)
    total1 = _tc_norm(reps0, reps1)
    reps2, _ = _sc_layer2(deg_tab, reps1, rows_w, cols_w, zeros_tab)
    total = _tc_norm(total1, reps2)

    return total[:U], total[P:P + I]


# speedup vs baseline: 54.7336x; 1.0156x over previous
import functools

import jax
import jax.numpy as jnp
from jax import lax
from jax.experimental import pallas as pl
from jax.experimental.pallas import tpu as pltpu
from jax.experimental.pallas import tpu_sc as plsc

U = 25000
I = 25000
EMB = 32
E = 800000

P = 25088
NTAB = 2 * P
NSUB = 16
CHUNK = 128
NCHUNK = 392
EPT = CHUNK * NCHUNK
EPC = NSUB * EPT
ROWS_PT = P // NSUB
PAD_IDX = 25080
DEGW = 16
BLK = 28
NBLK = NCHUNK // BLK
NBUF = 14
DCH = 112
NDCH = ROWS_PT // DCH


def _sc_mesh():
    return plsc.VectorSubcoreMesh(core_axis_name="c", subcore_axis_name="s")


_SC_PARAMS = pltpu.CompilerParams(
    use_tc_tiling_on_sc=False, needs_layout_passes=False)


def _rsqrt16(d):
    i = plsc.bitcast(d, jnp.int32)
    i = jnp.int32(0x5F3759DF) - lax.shift_right_logical(i, 1)
    y = plsc.bitcast(i, jnp.float32)
    for _ in range(3):
        y = y * (1.5 - 0.5 * d * y * y)
    return y


@functools.partial(
    pl.kernel,
    out_type=pltpu.HBM((NTAB, DEGW), jnp.float32),
    mesh=_sc_mesh(),
    scratch_types=[
        pltpu.VMEM((NCHUNK, CHUNK), jnp.int32),
        pltpu.VMEM((CHUNK, DEGW), jnp.float32),
        pltpu.VMEM((DCH, DEGW), jnp.float32),
        pltpu.VMEM_SHARED((P, DEGW), jnp.float32),
        pltpu.SemaphoreType.DMA,
    ],
    compiler_params=_SC_PARAMS,
)
def _sc_degrees(rows_hbm, ones_hbm, zeros_hbm, out_hbm, ridx, ones_v, dchunk,
                deg_sh, sem):
    c = lax.axis_index("c")
    s = lax.axis_index("s")
    w = c * NSUB + s
    base = s * ROWS_PT
    pltpu.sync_copy(zeros_hbm.at[pl.ds(base, ROWS_PT)],
                    deg_sh.at[pl.ds(base, ROWS_PT)])
    pltpu.sync_copy(ones_hbm, ones_v)
    pltpu.sync_copy(rows_hbm.at[pl.ds(w * NCHUNK, NCHUNK)], ridx)
    plsc.subcore_barrier()

    def body(j, carry):
        pltpu.async_copy(ones_v, deg_sh.at[ridx.at[j]], sem, add=True)

        @pl.when(j >= 3)
        def _():
            pltpu.make_async_copy(ones_v, deg_sh.at[ridx.at[j]], sem).wait()

        return carry

    lax.fori_loop(0, NCHUNK, body, 0)
    for _ in range(3):
        pltpu.make_async_copy(ones_v, deg_sh.at[ridx.at[0]], sem).wait()
    plsc.subcore_barrier()

    def cchunk(i, carry):
        rl = base + i * DCH
        pltpu.sync_copy(deg_sh.at[pl.ds(rl, DCH)], dchunk)

        def rowf(r, carry2):
            dchunk[r, pl.ds(0, DEGW)] = _rsqrt16(dchunk[r, pl.ds(0, DEGW)])
            return carry2

        lax.fori_loop(0, DCH, rowf, 0)
        pltpu.sync_copy(dchunk, out_hbm.at[pl.ds(c * P + rl, DCH)])
        return carry

    lax.fori_loop(0, NDCH, cchunk, 0)


def _make_sc_layer(inv):
    @functools.partial(
        pl.kernel,
        out_type=[
            pltpu.HBM((NTAB, EMB), jnp.float32),
            pltpu.HBM((NTAB, EMB), jnp.float32),
        ],
        mesh=_sc_mesh(),
        scratch_types=[
            pltpu.VMEM((BLK, CHUNK), jnp.int32),
            pltpu.VMEM((BLK, CHUNK), jnp.int32),
            [pltpu.VMEM((CHUNK, EMB), jnp.float32) for _ in range(NBUF)],
            pltpu.VMEM((DCH, EMB), jnp.float32),
            pltpu.VMEM((DCH, DEGW), jnp.float32),
            pltpu.VMEM((DCH, EMB), jnp.float32),
            pltpu.VMEM_SHARED((P, EMB), jnp.float32),
            pltpu.SemaphoreType.DMA,
            pltpu.SemaphoreType.DMA,
        ],
        compiler_params=_SC_PARAMS,
    )
    def _sc_layer(s_hbm, prev_hbm, rows_hbm, cols_hbm, zeros_hbm,
                  reps_out, tmp_hbm,
                  ridx, cidx, gbufs, pbuf, dbuf, tbuf, acc_sh, sem, sem2):
        c = lax.axis_index("c")
        s = lax.axis_index("s")
        w = c * NSUB + s
        base = s * ROWS_PT
        own0 = c * P + base
        oth0 = (1 - c) * P + base
        pltpu.sync_copy(zeros_hbm.at[pl.ds(base, ROWS_PT)],
                        acc_sh.at[pl.ds(base, ROWS_PT)])

        def p1(i, carry):
            rg = oth0 + i * DCH
            pltpu.sync_copy(prev_hbm.at[pl.ds(rg, DCH)], pbuf)
            pltpu.sync_copy(s_hbm.at[pl.ds(rg, DCH)], dbuf)

            def rowf(h, carry2):
                for u in range(2):
                    r = h * 2 + u
                    sv = dbuf[r, pl.ds(0, 16)]
                    tbuf[r, pl.ds(0, 16)] = pbuf[r, pl.ds(0, 16)] * sv
                    tbuf[r, pl.ds(16, 16)] = pbuf[r, pl.ds(16, 16)] * sv
                return carry2

            lax.fori_loop(0, DCH // 2, rowf, 0)
            pltpu.sync_copy(tbuf, tmp_hbm.at[pl.ds(c * P + base + i * DCH, DCH)])
            return carry

        lax.fori_loop(0, NDCH, p1, 0)
        plsc.subcore_barrier()

        GA = NBUF - 2

        def blk_body(b, carry):
            off = w * NCHUNK + b * BLK
            pltpu.sync_copy(rows_hbm.at[pl.ds(off, BLK)], ridx)
            pltpu.sync_copy(cols_hbm.at[pl.ds(off, BLK)], cidx)
            for k in range(GA):
                pltpu.async_copy(tmp_hbm.at[cidx.at[k]], gbufs[k], sem)

            def grp_body(q, carry2):
                for k in range(NBUF):
                    j = q * NBUF + k
                    pltpu.make_async_copy(
                        tmp_hbm.at[cidx.at[j]], gbufs[k], sem).wait()
                    pltpu.async_copy(
                        gbufs[k], acc_sh.at[ridx.at[j]], sem2, add=True)

                    @pl.when(j >= 2)
                    def _():
                        pltpu.make_async_copy(
                            gbufs[k], acc_sh.at[ridx.at[j]], sem2).wait()

                    @pl.when(j + GA < BLK)
                    def _():
                        pltpu.async_copy(
                            tmp_hbm.at[cidx.at[j + GA]],
                            gbufs[(k + GA) % NBUF], sem)
                return carry2

            lax.fori_loop(0, BLK // NBUF, grp_body, 0)
            for _ in range(2):
                pltpu.make_async_copy(
                    gbufs[0], acc_sh.at[ridx.at[0]], sem2).wait()
            return carry

        lax.fori_loop(0, NBLK, blk_body, 0)
        plsc.subcore_barrier()

        def p3(i, carry):
            rl = base + i * DCH
            rg = own0 + i * DCH
            pltpu.sync_copy(acc_sh.at[pl.ds(rl, DCH)], pbuf)
            pltpu.sync_copy(s_hbm.at[pl.ds(rg, DCH)], dbuf)

            def rowf(h, carry2):
                for u in range(2):
                    r = h * 2 + u
                    sv = dbuf[r, pl.ds(0, 16)] * inv
                    tbuf[r, pl.ds(0, 16)] = pbuf[r, pl.ds(0, 16)] * sv
                    tbuf[r, pl.ds(16, 16)] = pbuf[r, pl.ds(16, 16)] * sv
                return carry2

            lax.fori_loop(0, DCH // 2, rowf, 0)
            pltpu.sync_copy(tbuf, reps_out.at[pl.ds(rg, DCH)])
            return carry

        lax.fori_loop(0, NDCH, p3, 0)

    return _sc_layer


_sc_layer1 = _make_sc_layer(0.5)
_sc_layer2 = _make_sc_layer(1.0 / 3.0)


TC_X = NTAB // 4
TC_BS = TC_X // 4
_TC_GRID = 4


def _tc_norm_body(acc_ref, reps_ref, seg_ref, out_ref):
    v = reps_ref[:]
    s2 = jnp.dot(v * v, seg_ref[:], preferred_element_type=jnp.float32,
                 precision=lax.Precision.HIGHEST)
    out_ref[:] = acc_ref[:] + v / jnp.maximum(jnp.sqrt(s2), 1e-12)


def _tc_norm(acc, reps, seg):
    out = pl.pallas_call(
        _tc_norm_body,
        grid=(_TC_GRID,),
        in_specs=[
            pl.BlockSpec((TC_BS, 128), lambda i: (i, 0)),
            pl.BlockSpec((TC_BS, 128), lambda i: (i, 0)),
            pl.BlockSpec((128, 128), lambda i: (0, 0)),
        ],
        out_specs=pl.BlockSpec((TC_BS, 128), lambda i: (i, 0)),
        out_shape=jax.ShapeDtypeStruct((TC_X, 128), jnp.float32),
    )(acc.reshape(TC_X, 128), reps.reshape(TC_X, 128), seg)
    return out.reshape(NTAB, EMB)


def _pad_edges(x):
    return jnp.concatenate(
        [x, jnp.full((EPC - E,), PAD_IDX, dtype=jnp.int32)])


def kernel(users_rep, items_rep, edge_index):
    src = edge_index[0].astype(jnp.int32)
    dst = edge_index[1].astype(jnp.int32)

    rows_w = jnp.concatenate([
        _pad_edges(src).reshape(NSUB * NCHUNK, CHUNK),
        _pad_edges(dst).reshape(NSUB * NCHUNK, CHUNK),
    ], axis=0)
    cols_w = jnp.concatenate([
        _pad_edges(dst).reshape(NSUB * NCHUNK, CHUNK),
        _pad_edges(src + P).reshape(NSUB * NCHUNK, CHUNK),
    ], axis=0)

    zpad = jnp.zeros((P - U, EMB), jnp.float32)
    reps0 = jnp.concatenate([users_rep, zpad, items_rep, zpad], axis=0)

    ones_hbm = jnp.ones((CHUNK, DEGW), jnp.float32)
    zeros_deg = jnp.zeros((P, DEGW), jnp.float32)
    zeros_tab = jnp.zeros((P, EMB), jnp.float32)

    seg = (jnp.arange(128)[:, None] // EMB
           == jnp.arange(128)[None, :] // EMB).astype(jnp.float32)

    s_tab = _sc_degrees(rows_w, ones_hbm, zeros_deg)

    reps1, _ = _sc_layer1(s_tab, reps0, rows_w, cols_w, zeros_tab)
    total1 = _tc_norm(reps0, reps1, seg)
    reps2, _ = _sc_layer2(s_tab, reps1, rows_w, cols_w, zeros_tab)
    total = _tc_norm(total1, reps2, seg)

    return total[:U], total[P:P + I]


# speedup vs baseline: 59.2765x; 1.0830x over previous
import functools

import jax
import jax.numpy as jnp
from jax import lax
from jax.experimental import pallas as pl
from jax.experimental.pallas import tpu as pltpu
from jax.experimental.pallas import tpu_sc as plsc

U = 25000
I = 25000
EMB = 32
E = 800000

P = 25088
NTAB = 2 * P
NSUB = 16
CHUNK = 128
NCHUNK = 392
EPT = CHUNK * NCHUNK
EPC = NSUB * EPT
ROWS_PT = P // NSUB
PAD_IDX = 25080
DEGW = 16
BLK = 56
NBLK = NCHUNK // BLK
NBUF = 8
DCH = 224
NDCH = ROWS_PT // DCH


def _sc_mesh():
    return plsc.VectorSubcoreMesh(core_axis_name="c", subcore_axis_name="s")


_SC_PARAMS = pltpu.CompilerParams(
    use_tc_tiling_on_sc=False, needs_layout_passes=False)


def _rsqrt16(d):
    i = plsc.bitcast(d, jnp.int32)
    i = jnp.int32(0x5F3759DF) - lax.shift_right_logical(i, 1)
    y = plsc.bitcast(i, jnp.float32)
    for _ in range(3):
        y = y * (1.5 - 0.5 * d * y * y)
    return y


@functools.partial(
    pl.kernel,
    out_type=pltpu.HBM((NTAB, DEGW), jnp.float32),
    mesh=_sc_mesh(),
    scratch_types=[
        pltpu.VMEM((NCHUNK, CHUNK), jnp.int32),
        pltpu.VMEM((CHUNK, DEGW), jnp.float32),
        pltpu.VMEM((DCH, DEGW), jnp.float32),
        pltpu.VMEM_SHARED((P, DEGW), jnp.float32),
        pltpu.SemaphoreType.DMA,
    ],
    compiler_params=_SC_PARAMS,
)
def _sc_degrees(rows_hbm, ones_hbm, zeros_hbm, out_hbm, ridx, ones_v, dchunk,
                deg_sh, sem):
    c = lax.axis_index("c")
    s = lax.axis_index("s")
    w = c * NSUB + s
    base = s * ROWS_PT
    pltpu.sync_copy(zeros_hbm.at[pl.ds(base, ROWS_PT)],
                    deg_sh.at[pl.ds(base, ROWS_PT)])
    pltpu.sync_copy(ones_hbm, ones_v)
    pltpu.sync_copy(rows_hbm.at[pl.ds(w * NCHUNK, NCHUNK)], ridx)
    plsc.subcore_barrier()

    def body(j, carry):
        pltpu.async_copy(ones_v, deg_sh.at[ridx.at[j]], sem, add=True)

        @pl.when(j >= 3)
        def _():
            pltpu.make_async_copy(ones_v, deg_sh.at[ridx.at[j]], sem).wait()

        return carry

    lax.fori_loop(0, NCHUNK, body, 0)
    for _ in range(3):
        pltpu.make_async_copy(ones_v, deg_sh.at[ridx.at[0]], sem).wait()
    plsc.subcore_barrier()

    def cchunk(i, carry):
        rl = base + i * DCH
        pltpu.sync_copy(deg_sh.at[pl.ds(rl, DCH)], dchunk)

        def rowf(r, carry2):
            dchunk[r, pl.ds(0, DEGW)] = _rsqrt16(dchunk[r, pl.ds(0, DEGW)])
            return carry2

        lax.fori_loop(0, DCH, rowf, 0)
        pltpu.sync_copy(dchunk, out_hbm.at[pl.ds(c * P + rl, DCH)])
        return carry

    lax.fori_loop(0, NDCH, cchunk, 0)


def _make_sc_layer(inv):
    @functools.partial(
        pl.kernel,
        out_type=[
            pltpu.HBM((NTAB, EMB), jnp.float32),
            pltpu.HBM((NTAB, EMB), jnp.float32),
        ],
        mesh=_sc_mesh(),
        scratch_types=[
            pltpu.VMEM((BLK, CHUNK), jnp.int32),
            pltpu.VMEM((BLK, CHUNK), jnp.int32),
            [pltpu.VMEM((CHUNK, EMB), jnp.float32) for _ in range(NBUF)],
            pltpu.VMEM((DCH, EMB), jnp.float32),
            pltpu.VMEM((DCH, DEGW), jnp.float32),
            pltpu.VMEM((DCH, EMB), jnp.float32),
            pltpu.VMEM_SHARED((P, EMB), jnp.float32),
            pltpu.SemaphoreType.DMA,
            pltpu.SemaphoreType.DMA,
        ],
        compiler_params=_SC_PARAMS,
    )
    def _sc_layer(s_hbm, prev_hbm, rows_hbm, cols_hbm, zeros_hbm,
                  reps_out, tmp_hbm,
                  ridx, cidx, gbufs, pbuf, dbuf, tbuf, acc_sh, sem, sem2):
        c = lax.axis_index("c")
        s = lax.axis_index("s")
        w = c * NSUB + s
        base = s * ROWS_PT
        own0 = c * P + base
        oth0 = (1 - c) * P + base
        pltpu.sync_copy(zeros_hbm.at[pl.ds(base, ROWS_PT)],
                        acc_sh.at[pl.ds(base, ROWS_PT)])

        def p1(i, carry):
            rg = oth0 + i * DCH
            pltpu.sync_copy(prev_hbm.at[pl.ds(rg, DCH)], pbuf)
            pltpu.sync_copy(s_hbm.at[pl.ds(rg, DCH)], dbuf)

            def rowf(h, carry2):
                for u in range(2):
                    r = h * 2 + u
                    sv = dbuf[r, pl.ds(0, 16)]
                    tbuf[r, pl.ds(0, 16)] = pbuf[r, pl.ds(0, 16)] * sv
                    tbuf[r, pl.ds(16, 16)] = pbuf[r, pl.ds(16, 16)] * sv
                return carry2

            lax.fori_loop(0, DCH // 2, rowf, 0)
            pltpu.sync_copy(tbuf, tmp_hbm.at[pl.ds(c * P + base + i * DCH, DCH)])
            return carry

        lax.fori_loop(0, NDCH, p1, 0)
        plsc.subcore_barrier()

        GA = NBUF - 2

        def blk_body(b, carry):
            off = w * NCHUNK + b * BLK
            pltpu.sync_copy(rows_hbm.at[pl.ds(off, BLK)], ridx)
            pltpu.sync_copy(cols_hbm.at[pl.ds(off, BLK)], cidx)
            for k in range(GA):
                pltpu.async_copy(tmp_hbm.at[cidx.at[k]], gbufs[k], sem)

            def grp_body(q, carry2):
                for k in range(NBUF):
                    j = q * NBUF + k
                    pltpu.make_async_copy(
                        tmp_hbm.at[cidx.at[j]], gbufs[k], sem).wait()
                    pltpu.async_copy(
                        gbufs[k], acc_sh.at[ridx.at[j]], sem2, add=True)

                    @pl.when(j >= 2)
                    def _():
                        pltpu.make_async_copy(
                            gbufs[k], acc_sh.at[ridx.at[j]], sem2).wait()

                    @pl.when(j + GA < BLK)
                    def _():
                        pltpu.async_copy(
                            tmp_hbm.at[cidx.at[j + GA]],
                            gbufs[(k + GA) % NBUF], sem)
                return carry2

            lax.fori_loop(0, BLK // NBUF, grp_body, 0)
            for _ in range(2):
                pltpu.make_async_copy(
                    gbufs[0], acc_sh.at[ridx.at[0]], sem2).wait()
            return carry

        lax.fori_loop(0, NBLK, blk_body, 0)
        plsc.subcore_barrier()

        def p3(i, carry):
            rl = base + i * DCH
            rg = own0 + i * DCH
            pltpu.sync_copy(acc_sh.at[pl.ds(rl, DCH)], pbuf)
            pltpu.sync_copy(s_hbm.at[pl.ds(rg, DCH)], dbuf)

            def rowf(h, carry2):
                for u in range(2):
                    r = h * 2 + u
                    sv = dbuf[r, pl.ds(0, 16)] * inv
                    tbuf[r, pl.ds(0, 16)] = pbuf[r, pl.ds(0, 16)] * sv
                    tbuf[r, pl.ds(16, 16)] = pbuf[r, pl.ds(16, 16)] * sv
                return carry2

            lax.fori_loop(0, DCH // 2, rowf, 0)
            pltpu.sync_copy(tbuf, reps_out.at[pl.ds(rg, DCH)])
            return carry

        lax.fori_loop(0, NDCH, p3, 0)

    return _sc_layer


_sc_layer1 = _make_sc_layer(0.5)
_sc_layer2 = _make_sc_layer(1.0 / 3.0)


TC_X = NTAB // 4
TC_BS = TC_X // 4
_TC_GRID = 4


def _tc_norm_body(acc_ref, reps_ref, seg_ref, out_ref):
    v = reps_ref[:]
    s2 = jnp.dot(v * v, seg_ref[:], preferred_element_type=jnp.float32,
                 precision=lax.Precision.HIGHEST)
    out_ref[:] = acc_ref[:] + v / jnp.maximum(jnp.sqrt(s2), 1e-12)


def _tc_norm(acc, reps, seg):
    out = pl.pallas_call(
        _tc_norm_body,
        grid=(_TC_GRID,),
        in_specs=[
            pl.BlockSpec((TC_BS, 128), lambda i: (i, 0)),
            pl.BlockSpec((TC_BS, 128), lambda i: (i, 0)),
            pl.BlockSpec((128, 128), lambda i: (0, 0)),
        ],
        out_specs=pl.BlockSpec((TC_BS, 128), lambda i: (i, 0)),
        out_shape=jax.ShapeDtypeStruct((TC_X, 128), jnp.float32),
    )(acc.reshape(TC_X, 128), reps.reshape(TC_X, 128), seg)
    return out.reshape(NTAB, EMB)


def _pad_edges(x):
    return jnp.concatenate(
        [x, jnp.full((EPC - E,), PAD_IDX, dtype=jnp.int32)])


def kernel(users_rep, items_rep, edge_index):
    src = edge_index[0].astype(jnp.int32)
    dst = edge_index[1].astype(jnp.int32)

    rows_w = jnp.concatenate([
        _pad_edges(src).reshape(NSUB * NCHUNK, CHUNK),
        _pad_edges(dst).reshape(NSUB * NCHUNK, CHUNK),
    ], axis=0)
    cols_w = jnp.concatenate([
        _pad_edges(dst).reshape(NSUB * NCHUNK, CHUNK),
        _pad_edges(src + P).reshape(NSUB * NCHUNK, CHUNK),
    ], axis=0)

    zpad = jnp.zeros((P - U, EMB), jnp.float32)
    reps0 = jnp.concatenate([users_rep, zpad, items_rep, zpad], axis=0)

    ones_hbm = jnp.ones((CHUNK, DEGW), jnp.float32)
    zeros_deg = jnp.zeros((P, DEGW), jnp.float32)
    zeros_tab = jnp.zeros((P, EMB), jnp.float32)

    seg = (jnp.arange(128)[:, None] // EMB
           == jnp.arange(128)[None, :] // EMB).astype(jnp.float32)

    s_tab = _sc_degrees(rows_w, ones_hbm, zeros_deg)

    reps1, _ = _sc_layer1(s_tab, reps0, rows_w, cols_w, zeros_tab)
    total1 = _tc_norm(reps0, reps1, seg)
    reps2, _ = _sc_layer2(s_tab, reps1, rows_w, cols_w, zeros_tab)
    total = _tc_norm(total1, reps2, seg)

    return total[:U], total[P:P + I]
